# Initial kernel scaffold; baseline (speedup 1.0000x reference)
import jax
import jax.numpy as jnp
from jax.experimental import pallas as pl


def kernel(high_x, high_edge_index, low_x, low_edge_index, mlp_high_W, mlp_high_b, mlp_low_W, mlp_low_b, Wh, ash, adh, Wl, asl, adl, ln_gamma, ln_beta):
    raise NotImplementedError("write your pallas kernel here")


# speedup vs baseline: 80.1215x; 80.1215x over previous
import functools

import jax
import jax.numpy as jnp
from jax import lax
from jax.experimental import pallas as pl
from jax.experimental.pallas import tpu as pltpu
from jax.experimental.pallas import tpu_sc as plsc

NN = 100000
EE = 1600000
NH = NN // 2
NC, NS = 2, 16
NW = NC * NS
EPT = EE // NW
RPT = NN // NS
BA = 400
BM = 400
BN = 2000
F32 = jnp.float32

_HIGHEST = jax.lax.Precision.HIGHEST


def _dot(a, b):
    return jnp.dot(a, b, precision=_HIGHEST, preferred_element_type=F32)


_GDN = lax.GatherDimensionNumbers(offset_dims=(), collapsed_slice_dims=(0,),
                                  start_index_map=(0,))


def _take16(x, idx):
    return lax.gather(x, idx[:, None], _GDN, slice_sizes=(1,),
                      mode=lax.GatherScatterMode.PROMISE_IN_BOUNDS)


def _tc_embed_body(x_ref, bd0_ref, b_ref, bd1_ref, aed_ref,
                   emb_ref, hw_ref, esed_ref):
    x = x_ref[...]
    emb = jax.nn.gelu(_dot(x, bd0_ref[...]) + b_ref[...])
    hw = _dot(emb, bd1_ref[...])
    emb_ref[...] = emb
    hw_ref[...] = hw
    esed_ref[...] = _dot(hw, aed_ref[...])


def _tc_mid_body(op_ref, dp_ref, embp_ref, p_ref, ex_ref, bd_ref, aed_ref,
                 emb_ref, hw_ref, esed_ref, *, residual):
    op = op_ref[...]
    m = op[0] + op[1]
    num = _dot(m[0], p_ref[0])
    for h in range(1, 4):
        num = num + _dot(m[h], p_ref[h])
    dp = dp_ref[...]
    den = _dot(dp[0] + dp[1], ex_ref[...])
    neu = jax.nn.gelu(num / (den + 1e-9))
    emb = neu + embp_ref[...] if residual else neu
    hw = _dot(emb, bd_ref[...])
    emb_ref[...] = emb
    hw_ref[...] = hw
    esed_ref[...] = _dot(hw, aed_ref[...])


def _tc_final_body(op_ref, dp_ref, embp_ref, p_ref, ex_ref, m64_ref,
                   g_ref, be_ref, y_ref):
    op = op_ref[...]
    m = op[0] + op[1]
    num = _dot(m[0], p_ref[0])
    for h in range(1, 4):
        num = num + _dot(m[h], p_ref[h])
    dp = dp_ref[...]
    den = _dot(dp[0] + dp[1], ex_ref[...])
    emb = jax.nn.gelu(num / (den + 1e-9)) + embp_ref[...]
    mu = _dot(emb, m64_ref[...])
    d = emb - mu
    v = _dot(d * d, m64_ref[...])
    y_ref[...] = d / jnp.sqrt(v + 1e-5) * g_ref[...] + be_ref[...]


def _full_spec(shape):
    nd = len(shape)
    return pl.BlockSpec(shape, lambda i, _n=nd: (0,) * _n)


def _tc_embed(x2, bd0, b128, bd1, aed):
    k2 = x2.shape[1]
    grid = NH // BN
    return pl.pallas_call(
        _tc_embed_body,
        grid=(grid,),
        in_specs=[
            pl.BlockSpec((BN, k2), lambda i: (i, 0)),
            _full_spec((k2, 128)),
            _full_spec((1, 128)),
            _full_spec((128, 128)),
            _full_spec((128, 64)),
        ],
        out_specs=[
            pl.BlockSpec((BN, 128), lambda i: (i, 0)),
            pl.BlockSpec((BN, 128), lambda i: (i, 0)),
            pl.BlockSpec((BN, 64), lambda i: (i, 0)),
        ],
        out_shape=[
            jax.ShapeDtypeStruct((NH, 128), F32),
            jax.ShapeDtypeStruct((NH, 128), F32),
            jax.ShapeDtypeStruct((NH, 64), F32),
        ],
    )(x2, bd0, b128, bd1, aed)


def _tc_mid(op, dp, embp, pst, ex, bd, aed, residual):
    grid = NH // BN
    return pl.pallas_call(
        functools.partial(_tc_mid_body, residual=residual),
        grid=(grid,),
        in_specs=[
            pl.BlockSpec((2, 4, BN, 32), lambda i: (0, 0, i, 0)),
            pl.BlockSpec((2, BN, 32), lambda i: (0, i, 0)),
            pl.BlockSpec((BN, 128), lambda i: (i, 0)),
            _full_spec((4, 32, 128)),
            _full_spec((32, 128)),
            _full_spec((128, 128)),
            _full_spec((128, 64)),
        ],
        out_specs=[
            pl.BlockSpec((BN, 128), lambda i: (i, 0)),
            pl.BlockSpec((BN, 128), lambda i: (i, 0)),
            pl.BlockSpec((BN, 64), lambda i: (i, 0)),
        ],
        out_shape=[
            jax.ShapeDtypeStruct((NH, 128), F32),
            jax.ShapeDtypeStruct((NH, 128), F32),
            jax.ShapeDtypeStruct((NH, 64), F32),
        ],
    )(op, dp, embp, pst, ex, bd, aed)


def _tc_final(op, dp, embp, pst, ex, m64, g128, b128):
    grid = NH // BN
    return pl.pallas_call(
        _tc_final_body,
        grid=(grid,),
        in_specs=[
            pl.BlockSpec((2, 4, BN, 32), lambda i: (0, 0, i, 0)),
            pl.BlockSpec((2, BN, 32), lambda i: (0, i, 0)),
            pl.BlockSpec((BN, 128), lambda i: (i, 0)),
            _full_spec((4, 32, 128)),
            _full_spec((32, 128)),
            _full_spec((128, 128)),
            _full_spec((1, 128)),
            _full_spec((1, 128)),
        ],
        out_specs=pl.BlockSpec((BN, 128), lambda i: (i, 0)),
        out_shape=jax.ShapeDtypeStruct((NH, 128), F32),
    )(op, dp, embp, pst, ex, m64, g128, b128)


_MESH = plsc.VectorSubcoreMesh(core_axis_name="c", subcore_axis_name="s",
                               num_cores=NC, num_subcores=NS)
_DEN_PAD = 15 * RPT + 16 * BA


def _attn_body(src_hbm, dst_hbm, esed_hbm, w_hbm, denp_hbm,
               den_s, src_v, dst_v, es_v, ed_v, w_v, w4_v):
    cid = lax.axis_index("c")
    sid = lax.axis_index("s")
    wid = sid * NC + cid
    ridx = lax.iota(jnp.int32, 16) // 4
    cidx = lax.iota(jnp.int32, 16) % 4
    zero16 = jnp.zeros((16,), F32)

    def _zw(j, c):
        w_v[j, :] = zero16
        return c
    lax.fori_loop(0, BA, _zw, 0)
    for t in range(16):
        pltpu.sync_copy(w_v, den_s.at[pl.ds(sid * RPT + t * BA, BA), :])
    plsc.subcore_barrier()

    def _blk(b, c):
        eb = wid * EPT + b * BA
        pltpu.sync_copy(src_hbm.at[pl.ds(eb, BA)], src_v)
        pltpu.sync_copy(dst_hbm.at[pl.ds(eb, BA)], dst_v)

        def _mkidx(g, cc):
            sl = pl.ds(16 * g, 16)
            src_v[sl] = 2 * src_v[sl]
            dst_v[sl] = 2 * dst_v[sl] + 1
            return cc
        lax.fori_loop(0, BA // 16, _mkidx, 0)

        pltpu.sync_copy(esed_hbm.at[src_v], es_v)
        pltpu.sync_copy(esed_hbm.at[dst_v], ed_v)

        def _cmp(g, cc):
            r = ridx + 4 * g
            e = (plsc.load_gather(es_v, [r, cidx])
                 + plsc.load_gather(ed_v, [r, cidx]))
            e = jnp.where(e >= 0.0, e, 0.2 * e)
            w = jnp.exp(e)
            plsc.store_scatter(w_v, [r, cidx], w)
            plsc.store_scatter(w4_v, [r, cidx], w)
            return cc
        lax.fori_loop(0, BA // 4, _cmp, 0)

        def _undst(g, cc):
            sl = pl.ds(16 * g, 16)
            dst_v[sl] = lax.shift_right_logical(dst_v[sl], 1)
            return cc
        lax.fori_loop(0, BA // 16, _undst, 0)

        pltpu.sync_copy(w4_v, w_hbm.at[pl.ds(eb, BA), :])
        pltpu.sync_copy(w_v, den_s.at[dst_v], add=True)
        return c
    lax.fori_loop(0, EPT // BA, _blk, 0)

    plsc.subcore_barrier()
    fst = pl.multiple_of(sid * RPT - 2 * (sid % 4), 8)
    pltpu.sync_copy(den_s.at[pl.ds(fst, RPT + 6), :],
                    denp_hbm.at[cid, pl.ds(fst, RPT + 6), :])


_attn_w = functools.partial(
    pl.kernel,
    out_type=[jax.ShapeDtypeStruct((EE, 4), F32),
              jax.ShapeDtypeStruct((NC, NN, 16), F32)],
    mesh=_MESH,
    compiler_params=pltpu.CompilerParams(needs_layout_passes=False, use_tc_tiling_on_sc=False),
    scratch_types=[
        pltpu.VMEM_SHARED((_DEN_PAD, 16), F32),
        pltpu.VMEM((BA,), jnp.int32),
        pltpu.VMEM((BA,), jnp.int32),
        pltpu.VMEM((BA, 16), F32),
        pltpu.VMEM((BA, 16), F32),
        pltpu.VMEM((BA, 16), F32),
        pltpu.VMEM((BA, 4), F32),
    ],
)(_attn_body)


_ACC_PAD = 15 * RPT + 16 * BM


def _msg_body(src_hbm, dst_hbm, w_hbm, hw4_hbm, outp_hbm,
              acc_s, src_v, dst_v, w_v, rows_v):
    cid = lax.axis_index("c")
    sid = lax.axis_index("s")
    wid = sid * NC + cid
    ridx = lax.iota(jnp.int32, 16) // 4
    cidx = lax.iota(jnp.int32, 16) % 4
    zero16 = jnp.zeros((16,), F32)

    for h in range(4):
        def _zz(j, c):
            rows_v[j, :] = zero16
            return c
        lax.fori_loop(0, BM, _zz, 0)
        for t in range(16):
            pltpu.sync_copy(rows_v, acc_s.at[pl.ds(sid * RPT + t * BM, BM), :])
        plsc.subcore_barrier()

        def _blk(b, c, _h=h):
            eb = wid * EPT + b * BM
            pltpu.sync_copy(src_hbm.at[pl.ds(eb, BM)], src_v)
            pltpu.sync_copy(dst_hbm.at[pl.ds(eb, BM)], dst_v)
            pltpu.sync_copy(w_hbm.at[pl.ds(eb, BM), :], w_v)

            def _mkidx(g, cc):
                sl = pl.ds(16 * g, 16)
                src_v[sl] = 4 * src_v[sl] + _h
                return cc
            lax.fori_loop(0, BM // 16, _mkidx, 0)

            pltpu.sync_copy(hw4_hbm.at[src_v], rows_v)

            def _cmp(g, cc):
                w16 = plsc.load_gather(w_v, [ridx + 4 * g, cidx])
                for k in range(4):
                    j = 4 * g + k
                    spl = _take16(w16, jnp.full((16,), 4 * k + _h, jnp.int32))
                    rows_v[j, :] = rows_v[j, :] * spl
                return cc
            lax.fori_loop(0, BM // 4, _cmp, 0)

            pltpu.sync_copy(rows_v, acc_s.at[dst_v], add=True)
            return c
        lax.fori_loop(0, EPT // BM, _blk, 0)

        plsc.subcore_barrier()
        fst = pl.multiple_of(sid * RPT - 2 * (sid % 4), 8)
        pltpu.sync_copy(acc_s.at[pl.ds(fst, RPT + 6), :],
                        outp_hbm.at[cid, h, pl.ds(fst, RPT + 6), :])
        if h < 3:
            plsc.subcore_barrier()


_msg = functools.partial(
    pl.kernel,
    out_type=jax.ShapeDtypeStruct((NC, 4, NN, 16), F32),
    mesh=_MESH,
    compiler_params=pltpu.CompilerParams(needs_layout_passes=False, use_tc_tiling_on_sc=False),
    scratch_types=[
        pltpu.VMEM_SHARED((_ACC_PAD, 16), F32),
        pltpu.VMEM((BM,), jnp.int32),
        pltpu.VMEM((BM,), jnp.int32),
        pltpu.VMEM((BM, 4), F32),
        pltpu.VMEM((BM, 16), F32),
    ],
)(_msg_body)


def _blockdiag2(w):
    k = w.shape[0]
    z = jnp.zeros((2 * k, 128), F32)
    return z.at[:k, :64].set(w).at[k:, 64:].set(w)


def _build_aed(a_s, a_d):
    base = jnp.stack([a_s, a_d], 0)
    t = jnp.einsum("thd,hk->hdtk", base, jnp.eye(4, dtype=F32))
    t64 = t.reshape(64, 2, 4)
    blk = jnp.zeros((64, 32), F32)
    blk = blk.at[:, 0:4].set(t64[:, 0]).at[:, 16:20].set(t64[:, 1])
    a = jnp.zeros((128, 64), F32)
    return a.at[:64, :32].set(blk).at[64:, 32:].set(blk)


def kernel(high_x, high_edge_index, low_x, low_edge_index, mlp_high_W,
           mlp_high_b, mlp_low_W, mlp_low_b, Wh, ash, adh, Wl, asl, adl,
           ln_gamma, ln_beta):
    e16 = jnp.concatenate([jnp.eye(4, dtype=F32), jnp.zeros((12, 4), F32)], 0)
    ex = jnp.kron(jnp.eye(2, dtype=F32), jnp.kron(e16, jnp.ones((1, 16), F32)))
    pst = jnp.stack([
        jnp.kron(jnp.eye(2, dtype=F32),
                 jnp.kron(jax.nn.one_hot(h, 4, dtype=F32)[None, :],
                          jnp.eye(16, dtype=F32)))
        for h in range(4)])
    m64 = jnp.kron(jnp.eye(2, dtype=F32), jnp.ones((64, 64), F32) / 64.0)
    g128 = jnp.tile(ln_gamma, 2)[None, :]
    b128 = jnp.tile(ln_beta, 2)[None, :]

    outs = []
    for x, eidx, w0, b0, wl, a_s, a_d in (
            (high_x, high_edge_index, mlp_high_W, mlp_high_b, Wh, ash, adh),
            (low_x, low_edge_index, mlp_low_W, mlp_low_b, Wl, asl, adl)):
        src, dst = eidx[0], eidx[1]
        x2 = x.reshape(NH, 2 * x.shape[1])
        bds = [_blockdiag2(wl[i]) for i in range(4)]
        aeds = [_build_aed(a_s[i], a_d[i]) for i in range(4)]
        emb, hw, esed = _tc_embed(x2, _blockdiag2(w0),
                                  jnp.tile(b0, 2)[None, :], bds[0], aeds[0])
        for i in range(4):
            w, denp = _attn_w(src, dst, esed.reshape(2 * NN, 16))
            outp = _msg(src, dst, w, hw.reshape(4 * NN, 16))
            op = outp.reshape(NC, 4, NH, 32)
            dp = denp.reshape(NC, NH, 32)
            if i < 3:
                emb, hw, esed = _tc_mid(op, dp, emb, pst, ex,
                                        bds[i + 1], aeds[i + 1],
                                        residual=(i > 0))
            else:
                y = _tc_final(op, dp, emb, pst, ex, m64, g128, b128)
        outs.append(y.reshape(NN, 64))
    return tuple(outs)


# speedup vs baseline: 97.7531x; 1.2201x over previous
import functools

import jax
import jax.numpy as jnp
from jax import lax
from jax.experimental import pallas as pl
from jax.experimental.pallas import tpu as pltpu
from jax.experimental.pallas import tpu_sc as plsc

NN = 100000
EE = 1600000
NH = NN // 2
NC, NS = 2, 16
NW = NC * NS
EPT = EE // NW
RPT = NN // NS
BA = 400
BM = 400
BN = 2000
F32 = jnp.float32

_HIGHEST = jax.lax.Precision.HIGHEST


def _dot(a, b):
    return jnp.dot(a, b, precision=_HIGHEST, preferred_element_type=F32)


_GDN = lax.GatherDimensionNumbers(offset_dims=(), collapsed_slice_dims=(0,),
                                  start_index_map=(0,))


def _take16(x, idx):
    return lax.gather(x, idx[:, None], _GDN, slice_sizes=(1,),
                      mode=lax.GatherScatterMode.PROMISE_IN_BOUNDS)


def _tc_embed_body(x_ref, bd0_ref, b_ref, bd1_ref, aed_ref,
                   emb_ref, hw_ref, esed_ref):
    x = x_ref[...]
    emb = jax.nn.gelu(_dot(x, bd0_ref[...]) + b_ref[...])
    hw = _dot(emb, bd1_ref[...])
    emb_ref[...] = emb
    hw_ref[...] = hw
    esed_ref[...] = _dot(hw, aed_ref[...])


def _tc_mid_body(op_ref, dp_ref, embp_ref, p_ref, ex_ref, bd_ref, aed_ref,
                 emb_ref, hw_ref, esed_ref, *, residual):
    op = op_ref[...]
    m = op[0] + op[1]
    num = _dot(m[0], p_ref[0])
    for h in range(1, 4):
        num = num + _dot(m[h], p_ref[h])
    dp = dp_ref[...]
    den = _dot(dp[0] + dp[1], ex_ref[...])
    neu = jax.nn.gelu(num / (den + 1e-9))
    emb = neu + embp_ref[...] if residual else neu
    hw = _dot(emb, bd_ref[...])
    emb_ref[...] = emb
    hw_ref[...] = hw
    esed_ref[...] = _dot(hw, aed_ref[...])


def _tc_final_body(op_ref, dp_ref, embp_ref, p_ref, ex_ref, m64_ref,
                   g_ref, be_ref, y_ref):
    op = op_ref[...]
    m = op[0] + op[1]
    num = _dot(m[0], p_ref[0])
    for h in range(1, 4):
        num = num + _dot(m[h], p_ref[h])
    dp = dp_ref[...]
    den = _dot(dp[0] + dp[1], ex_ref[...])
    emb = jax.nn.gelu(num / (den + 1e-9)) + embp_ref[...]
    mu = _dot(emb, m64_ref[...])
    d = emb - mu
    v = _dot(d * d, m64_ref[...])
    y_ref[...] = d / jnp.sqrt(v + 1e-5) * g_ref[...] + be_ref[...]


def _full_spec(shape):
    nd = len(shape)
    return pl.BlockSpec(shape, lambda i, _n=nd: (0,) * _n)


def _tc_embed(x2, bd0, b128, bd1, aed):
    k2 = x2.shape[1]
    grid = NH // BN
    return pl.pallas_call(
        _tc_embed_body,
        grid=(grid,),
        in_specs=[
            pl.BlockSpec((BN, k2), lambda i: (i, 0)),
            _full_spec((k2, 128)),
            _full_spec((1, 128)),
            _full_spec((128, 128)),
            _full_spec((128, 64)),
        ],
        out_specs=[
            pl.BlockSpec((BN, 128), lambda i: (i, 0)),
            pl.BlockSpec((BN, 128), lambda i: (i, 0)),
            pl.BlockSpec((BN, 64), lambda i: (i, 0)),
        ],
        out_shape=[
            jax.ShapeDtypeStruct((NH, 128), F32),
            jax.ShapeDtypeStruct((NH, 128), F32),
            jax.ShapeDtypeStruct((NH, 64), F32),
        ],
    )(x2, bd0, b128, bd1, aed)


def _tc_mid(op, dp, embp, pst, ex, bd, aed, residual):
    grid = NH // BN
    return pl.pallas_call(
        functools.partial(_tc_mid_body, residual=residual),
        grid=(grid,),
        in_specs=[
            pl.BlockSpec((2, 4, BN, 32), lambda i: (0, 0, i, 0)),
            pl.BlockSpec((2, BN, 16), lambda i: (0, i, 0)),
            pl.BlockSpec((BN, 128), lambda i: (i, 0)),
            _full_spec((4, 32, 128)),
            _full_spec((16, 128)),
            _full_spec((128, 128)),
            _full_spec((128, 64)),
        ],
        out_specs=[
            pl.BlockSpec((BN, 128), lambda i: (i, 0)),
            pl.BlockSpec((BN, 128), lambda i: (i, 0)),
            pl.BlockSpec((BN, 64), lambda i: (i, 0)),
        ],
        out_shape=[
            jax.ShapeDtypeStruct((NH, 128), F32),
            jax.ShapeDtypeStruct((NH, 128), F32),
            jax.ShapeDtypeStruct((NH, 64), F32),
        ],
    )(op, dp, embp, pst, ex, bd, aed)


def _tc_final(op, dp, embp, pst, ex, m64, g128, b128):
    grid = NH // BN
    return pl.pallas_call(
        _tc_final_body,
        grid=(grid,),
        in_specs=[
            pl.BlockSpec((2, 4, BN, 32), lambda i: (0, 0, i, 0)),
            pl.BlockSpec((2, BN, 16), lambda i: (0, i, 0)),
            pl.BlockSpec((BN, 128), lambda i: (i, 0)),
            _full_spec((4, 32, 128)),
            _full_spec((16, 128)),
            _full_spec((128, 128)),
            _full_spec((1, 128)),
            _full_spec((1, 128)),
        ],
        out_specs=pl.BlockSpec((BN, 128), lambda i: (i, 0)),
        out_shape=jax.ShapeDtypeStruct((NH, 128), F32),
    )(op, dp, embp, pst, ex, m64, g128, b128)


_MESH = plsc.VectorSubcoreMesh(core_axis_name="c", subcore_axis_name="s",
                               num_cores=NC, num_subcores=NS)
CH = 2000
NBK = CH // BA
_DEN_PAD = 15 * RPT + 4 * CH


def _attn_body(src_hbm, dst_hbm, esed_hbm, w_hbm, denp_hbm,
               den_s, src_c, dst_c, es_v, ed_v, w_c):
    cid = lax.axis_index("c")
    sid = lax.axis_index("s")
    wid = sid * NC + cid
    ridx = lax.iota(jnp.int32, 16) // 4
    cidx = lax.iota(jnp.int32, 16) % 4
    r8 = lax.iota(jnp.int32, 16) // 8
    c8 = lax.iota(jnp.int32, 16) % 8
    zero16 = jnp.zeros((16,), F32)

    def _zw(g, c):
        plsc.store_scatter(w_c, [r8 + 2 * g, c8], zero16)
        return c
    lax.fori_loop(0, CH // 2, _zw, 0)
    for t in range(4):
        pltpu.sync_copy(w_c, den_s.at[pl.ds(sid * RPT + t * CH, CH), :])
    plsc.subcore_barrier()

    def _chunk(q, c):
        eb = wid * EPT + q * CH
        pltpu.sync_copy(src_hbm.at[pl.ds(eb, CH)], src_c)
        pltpu.sync_copy(dst_hbm.at[pl.ds(eb, CH)], dst_c)

        def _mkidx(g, cc):
            sl = pl.ds(16 * g, 16)
            src_c[sl] = 2 * src_c[sl]
            dst_c[sl] = 2 * dst_c[sl] + 1
            return cc
        lax.fori_loop(0, CH // 16, _mkidx, 0)

        for b in range(NBK):
            pltpu.sync_copy(esed_hbm.at[src_c.at[pl.ds(b * BA, BA)]], es_v)
            pltpu.sync_copy(esed_hbm.at[dst_c.at[pl.ds(b * BA, BA)]], ed_v)

            def _cmp(g, cc, _b=b):
                r = ridx + 4 * g
                e = (plsc.load_gather(es_v, [r, cidx])
                     + plsc.load_gather(ed_v, [r, cidx]))
                e = jnp.where(e >= 0.0, e, 0.2 * e)
                plsc.store_scatter(w_c, [_b * BA + r, cidx], jnp.exp(e))
                return cc
            lax.fori_loop(0, BA // 4, _cmp, 0)

        def _undst(g, cc):
            sl = pl.ds(16 * g, 16)
            dst_c[sl] = lax.shift_right_logical(dst_c[sl], 1)
            return cc
        lax.fori_loop(0, CH // 16, _undst, 0)

        pltpu.sync_copy(w_c, w_hbm.at[pl.ds(eb, CH), :])
        pltpu.sync_copy(w_c, den_s.at[dst_c], add=True)
        return c
    lax.fori_loop(0, EPT // CH, _chunk, 0)

    plsc.subcore_barrier()
    fst = pl.multiple_of(sid * RPT - 2 * (sid % 4), 8)
    pltpu.sync_copy(den_s.at[pl.ds(fst, RPT + 6), :],
                    denp_hbm.at[cid, pl.ds(fst, RPT + 6), :])


_attn_w = functools.partial(
    pl.kernel,
    out_type=[jax.ShapeDtypeStruct((EE, 8), F32),
              jax.ShapeDtypeStruct((NC, NN, 8), F32)],
    mesh=_MESH,
    compiler_params=pltpu.CompilerParams(needs_layout_passes=False, use_tc_tiling_on_sc=False),
    scratch_types=[
        pltpu.VMEM_SHARED((_DEN_PAD, 8), F32),
        pltpu.VMEM((CH,), jnp.int32),
        pltpu.VMEM((CH,), jnp.int32),
        pltpu.VMEM((BA, 16), F32),
        pltpu.VMEM((BA, 16), F32),
        pltpu.VMEM((CH, 8), F32),
    ],
)(_attn_body)


BMM = 400
_ACC_PAD = 15 * RPT + 16 * BMM


def _msg_body(src_hbm, dst_hbm, w_hbm, hw4_hbm, outp_hbm,
              acc_s, src_c, dst_b, w_c, rows_v):
    cid = lax.axis_index("c")
    sid = lax.axis_index("s")
    wid = sid * NC + cid
    ridx = lax.iota(jnp.int32, 16) // 4
    cidx = lax.iota(jnp.int32, 16) % 4
    zero16 = jnp.zeros((16,), F32)

    for h in range(4):
        def _zz(j, c):
            rows_v[j, :] = zero16
            return c
        lax.fori_loop(0, BMM, _zz, 0)
        for t in range(16):
            pltpu.sync_copy(rows_v, acc_s.at[pl.ds(sid * RPT + t * BMM, BMM), :])
        plsc.subcore_barrier()

        def _chunk(q, c, _h=h):
            eb = wid * EPT + q * CH
            pltpu.sync_copy(src_hbm.at[pl.ds(eb, CH)], src_c)
            pltpu.sync_copy(w_hbm.at[pl.ds(eb, CH), :], w_c)

            def _mkidx(g, cc):
                sl = pl.ds(16 * g, 16)
                src_c[sl] = 4 * src_c[sl] + _h
                return cc
            lax.fori_loop(0, CH // 16, _mkidx, 0)

            for b in range(CH // BMM):
                pltpu.sync_copy(dst_hbm.at[pl.ds(eb + b * BMM, BMM)], dst_b)
                pltpu.sync_copy(hw4_hbm.at[src_c.at[pl.ds(b * BMM, BMM)]],
                                rows_v)

                def _cmp(g, cc, _b=b, _hh=_h):
                    w16 = plsc.load_gather(w_c, [_b * BMM + ridx + 4 * g, cidx])
                    for k in range(4):
                        j = 4 * g + k
                        spl = _take16(w16,
                                      jnp.full((16,), 4 * k + _hh, jnp.int32))
                        rows_v[j, :] = rows_v[j, :] * spl
                    return cc
                lax.fori_loop(0, BMM // 4, _cmp, 0)

                pltpu.sync_copy(rows_v, acc_s.at[dst_b], add=True)
            return c
        lax.fori_loop(0, EPT // CH, _chunk, 0)

        plsc.subcore_barrier()
        fst = pl.multiple_of(sid * RPT - 2 * (sid % 4), 8)
        pltpu.sync_copy(acc_s.at[pl.ds(fst, RPT + 6), :],
                        outp_hbm.at[cid, h, pl.ds(fst, RPT + 6), :])
        if h < 3:
            plsc.subcore_barrier()


_msg = functools.partial(
    pl.kernel,
    out_type=jax.ShapeDtypeStruct((NC, 4, NN, 16), F32),
    mesh=_MESH,
    compiler_params=pltpu.CompilerParams(needs_layout_passes=False, use_tc_tiling_on_sc=False),
    scratch_types=[
        pltpu.VMEM_SHARED((_ACC_PAD, 16), F32),
        pltpu.VMEM((CH,), jnp.int32),
        pltpu.VMEM((BMM,), jnp.int32),
        pltpu.VMEM((CH, 8), F32),
        pltpu.VMEM((BMM, 16), F32),
    ],
)(_msg_body)


def _blockdiag2(w):
    k = w.shape[0]
    z = jnp.zeros((2 * k, 128), F32)
    return z.at[:k, :64].set(w).at[k:, 64:].set(w)


def _build_aed(a_s, a_d):
    base = jnp.stack([a_s, a_d], 0)
    t = jnp.einsum("thd,hk->hdtk", base, jnp.eye(4, dtype=F32))
    t64 = t.reshape(64, 2, 4)
    blk = jnp.zeros((64, 32), F32)
    blk = blk.at[:, 0:4].set(t64[:, 0]).at[:, 16:20].set(t64[:, 1])
    a = jnp.zeros((128, 64), F32)
    return a.at[:64, :32].set(blk).at[64:, 32:].set(blk)


def kernel(high_x, high_edge_index, low_x, low_edge_index, mlp_high_W,
           mlp_high_b, mlp_low_W, mlp_low_b, Wh, ash, adh, Wl, asl, adl,
           ln_gamma, ln_beta):
    e8 = jnp.concatenate([jnp.eye(4, dtype=F32), jnp.zeros((4, 4), F32)], 0)
    ex = jnp.kron(jnp.eye(2, dtype=F32), jnp.kron(e8, jnp.ones((1, 16), F32)))
    pst = jnp.stack([
        jnp.kron(jnp.eye(2, dtype=F32),
                 jnp.kron(jax.nn.one_hot(h, 4, dtype=F32)[None, :],
                          jnp.eye(16, dtype=F32)))
        for h in range(4)])
    m64 = jnp.kron(jnp.eye(2, dtype=F32), jnp.ones((64, 64), F32) / 64.0)
    g128 = jnp.tile(ln_gamma, 2)[None, :]
    b128 = jnp.tile(ln_beta, 2)[None, :]

    outs = []
    for x, eidx, w0, b0, wl, a_s, a_d in (
            (high_x, high_edge_index, mlp_high_W, mlp_high_b, Wh, ash, adh),
            (low_x, low_edge_index, mlp_low_W, mlp_low_b, Wl, asl, adl)):
        src, dst = eidx[0], eidx[1]
        x2 = x.reshape(NH, 2 * x.shape[1])
        bds = [_blockdiag2(wl[i]) for i in range(4)]
        aeds = [_build_aed(a_s[i], a_d[i]) for i in range(4)]
        emb, hw, esed = _tc_embed(x2, _blockdiag2(w0),
                                  jnp.tile(b0, 2)[None, :], bds[0], aeds[0])
        for i in range(4):
            w, denp = _attn_w(src, dst, esed.reshape(2 * NN, 16))
            outp = _msg(src, dst, w, hw.reshape(4 * NN, 16))
            op = outp.reshape(NC, 4, NH, 32)
            dp = denp.reshape(NC, NH, 16)
            if i < 3:
                emb, hw, esed = _tc_mid(op, dp, emb, pst, ex,
                                        bds[i + 1], aeds[i + 1],
                                        residual=(i > 0))
            else:
                y = _tc_final(op, dp, emb, pst, ex, m64, g128, b128)
        outs.append(y.reshape(NN, 64))
    return tuple(outs)


# speedup vs baseline: 98.7639x; 1.0103x over previous
import functools

import jax
import jax.numpy as jnp
from jax import lax
from jax.experimental import pallas as pl
from jax.experimental.pallas import tpu as pltpu
from jax.experimental.pallas import tpu_sc as plsc

NN = 100000
EE = 1600000
NH = NN // 2
NC, NS = 2, 16
NW = NC * NS
EPT = EE // NW
RPT = NN // NS
BA = 400
BM = 400
BN = 2000
F32 = jnp.float32

_HIGHEST = jax.lax.Precision.HIGHEST


def _dot(a, b):
    return jnp.dot(a, b, precision=_HIGHEST, preferred_element_type=F32)


_GDN = lax.GatherDimensionNumbers(offset_dims=(), collapsed_slice_dims=(0,),
                                  start_index_map=(0,))


def _take16(x, idx):
    return lax.gather(x, idx[:, None], _GDN, slice_sizes=(1,),
                      mode=lax.GatherScatterMode.PROMISE_IN_BOUNDS)


def _tc_embed_body(x_ref, bd0_ref, b_ref, bd1_ref, aed_ref,
                   emb_ref, hw_ref, esed_ref):
    x = x_ref[...]
    emb = jax.nn.gelu(_dot(x, bd0_ref[...]) + b_ref[...])
    hw = _dot(emb, bd1_ref[...])
    emb_ref[...] = emb
    hw_ref[...] = hw
    esed_ref[...] = _dot(hw, aed_ref[...])


def _tc_mid_body(op_ref, dp_ref, embp_ref, p_ref, ex_ref, bd_ref, aed_ref,
                 emb_ref, hw_ref, esed_ref, *, residual):
    op = op_ref[...]
    m = op[0] + op[1]
    num = _dot(m[0], p_ref[0])
    for h in range(1, 4):
        num = num + _dot(m[h], p_ref[h])
    dp = dp_ref[...]
    den = _dot(dp[0] + dp[1], ex_ref[...])
    neu = jax.nn.gelu(num / (den + 1e-9))
    emb = neu + embp_ref[...] if residual else neu
    hw = _dot(emb, bd_ref[...])
    emb_ref[...] = emb
    hw_ref[...] = hw
    esed_ref[...] = _dot(hw, aed_ref[...])


def _tc_final_body(op_ref, dp_ref, embp_ref, p_ref, ex_ref, m64_ref,
                   g_ref, be_ref, y_ref):
    op = op_ref[...]
    m = op[0] + op[1]
    num = _dot(m[0], p_ref[0])
    for h in range(1, 4):
        num = num + _dot(m[h], p_ref[h])
    dp = dp_ref[...]
    den = _dot(dp[0] + dp[1], ex_ref[...])
    emb = jax.nn.gelu(num / (den + 1e-9)) + embp_ref[...]
    mu = _dot(emb, m64_ref[...])
    d = emb - mu
    v = _dot(d * d, m64_ref[...])
    y_ref[...] = d / jnp.sqrt(v + 1e-5) * g_ref[...] + be_ref[...]


def _full_spec(shape):
    nd = len(shape)
    return pl.BlockSpec(shape, lambda i, _n=nd: (0,) * _n)


def _tc_embed(x2, bd0, b128, bd1, aed):
    k2 = x2.shape[1]
    grid = NH // BN
    return pl.pallas_call(
        _tc_embed_body,
        grid=(grid,),
        in_specs=[
            pl.BlockSpec((BN, k2), lambda i: (i, 0)),
            _full_spec((k2, 128)),
            _full_spec((1, 128)),
            _full_spec((128, 128)),
            _full_spec((128, 64)),
        ],
        out_specs=[
            pl.BlockSpec((BN, 128), lambda i: (i, 0)),
            pl.BlockSpec((BN, 128), lambda i: (i, 0)),
            pl.BlockSpec((BN, 64), lambda i: (i, 0)),
        ],
        out_shape=[
            jax.ShapeDtypeStruct((NH, 128), F32),
            jax.ShapeDtypeStruct((NH, 128), F32),
            jax.ShapeDtypeStruct((NH, 64), F32),
        ],
    )(x2, bd0, b128, bd1, aed)


def _tc_mid(op, dp, embp, pst, ex, bd, aed, residual):
    grid = NH // BN
    return pl.pallas_call(
        functools.partial(_tc_mid_body, residual=residual),
        grid=(grid,),
        in_specs=[
            pl.BlockSpec((2, 4, BN, 32), lambda i: (0, 0, i, 0)),
            pl.BlockSpec((2, BN, 16), lambda i: (0, i, 0)),
            pl.BlockSpec((BN, 128), lambda i: (i, 0)),
            _full_spec((4, 32, 128)),
            _full_spec((16, 128)),
            _full_spec((128, 128)),
            _full_spec((128, 64)),
        ],
        out_specs=[
            pl.BlockSpec((BN, 128), lambda i: (i, 0)),
            pl.BlockSpec((BN, 128), lambda i: (i, 0)),
            pl.BlockSpec((BN, 64), lambda i: (i, 0)),
        ],
        out_shape=[
            jax.ShapeDtypeStruct((NH, 128), F32),
            jax.ShapeDtypeStruct((NH, 128), F32),
            jax.ShapeDtypeStruct((NH, 64), F32),
        ],
    )(op, dp, embp, pst, ex, bd, aed)


def _tc_final(op, dp, embp, pst, ex, m64, g128, b128):
    grid = NH // BN
    return pl.pallas_call(
        _tc_final_body,
        grid=(grid,),
        in_specs=[
            pl.BlockSpec((2, 4, BN, 32), lambda i: (0, 0, i, 0)),
            pl.BlockSpec((2, BN, 16), lambda i: (0, i, 0)),
            pl.BlockSpec((BN, 128), lambda i: (i, 0)),
            _full_spec((4, 32, 128)),
            _full_spec((16, 128)),
            _full_spec((128, 128)),
            _full_spec((1, 128)),
            _full_spec((1, 128)),
        ],
        out_specs=pl.BlockSpec((BN, 128), lambda i: (i, 0)),
        out_shape=jax.ShapeDtypeStruct((NH, 128), F32),
    )(op, dp, embp, pst, ex, m64, g128, b128)


_MESH = plsc.VectorSubcoreMesh(core_axis_name="c", subcore_axis_name="s",
                               num_cores=NC, num_subcores=NS)
CH = 2000
NBK = CH // BA
_DEN_PAD = 15 * RPT + 4 * CH


def _attn_body(src_hbm, dst_hbm, esed_hbm, w_hbm, denp_hbm,
               den_s, src_c, dst_c, es_v, ed_v, w_c):
    cid = lax.axis_index("c")
    sid = lax.axis_index("s")
    wid = sid * NC + cid
    ridx = lax.iota(jnp.int32, 16) // 4
    cidx = lax.iota(jnp.int32, 16) % 4
    r8 = lax.iota(jnp.int32, 16) // 8
    c8 = lax.iota(jnp.int32, 16) % 8
    zero16 = jnp.zeros((16,), F32)

    def _zw(g, c):
        plsc.store_scatter(w_c, [r8 + 2 * g, c8], zero16)
        return c
    lax.fori_loop(0, CH // 2, _zw, 0)
    for t in range(4):
        pltpu.sync_copy(w_c, den_s.at[pl.ds(sid * RPT + t * CH, CH), :])
    plsc.subcore_barrier()

    def _chunk(q, c):
        eb = wid * EPT + q * CH
        pltpu.sync_copy(src_hbm.at[pl.ds(eb, CH)], src_c)
        pltpu.sync_copy(dst_hbm.at[pl.ds(eb, CH)], dst_c)

        def _mkidx(g, cc):
            sl = pl.ds(16 * g, 16)
            src_c[sl] = 2 * src_c[sl]
            dst_c[sl] = 2 * dst_c[sl] + 1
            return cc
        lax.fori_loop(0, CH // 16, _mkidx, 0)

        for b in range(NBK):
            pltpu.sync_copy(esed_hbm.at[src_c.at[pl.ds(b * BA, BA)]], es_v)
            pltpu.sync_copy(esed_hbm.at[dst_c.at[pl.ds(b * BA, BA)]], ed_v)

            def _cmp(g, cc, _b=b):
                for u in range(4):
                    r = ridx + 16 * g + 4 * u
                    e = (plsc.load_gather(es_v, [r, cidx])
                         + plsc.load_gather(ed_v, [r, cidx]))
                    e = jnp.where(e >= 0.0, e, 0.2 * e)
                    plsc.store_scatter(w_c, [_b * BA + r, cidx], jnp.exp(e))
                return cc
            lax.fori_loop(0, BA // 16, _cmp, 0)

        def _undst(g, cc):
            sl = pl.ds(16 * g, 16)
            dst_c[sl] = lax.shift_right_logical(dst_c[sl], 1)
            return cc
        lax.fori_loop(0, CH // 16, _undst, 0)

        pltpu.sync_copy(w_c, w_hbm.at[pl.ds(eb, CH), :])
        pltpu.sync_copy(w_c, den_s.at[dst_c], add=True)
        return c
    lax.fori_loop(0, EPT // CH, _chunk, 0)

    plsc.subcore_barrier()
    fst = pl.multiple_of(sid * RPT - 2 * (sid % 4), 8)
    pltpu.sync_copy(den_s.at[pl.ds(fst, RPT + 6), :],
                    denp_hbm.at[cid, pl.ds(fst, RPT + 6), :])


_attn_w = functools.partial(
    pl.kernel,
    out_type=[jax.ShapeDtypeStruct((EE, 8), F32),
              jax.ShapeDtypeStruct((NC, NN, 8), F32)],
    mesh=_MESH,
    compiler_params=pltpu.CompilerParams(needs_layout_passes=False, use_tc_tiling_on_sc=False),
    scratch_types=[
        pltpu.VMEM_SHARED((_DEN_PAD, 8), F32),
        pltpu.VMEM((CH,), jnp.int32),
        pltpu.VMEM((CH,), jnp.int32),
        pltpu.VMEM((BA, 16), F32),
        pltpu.VMEM((BA, 16), F32),
        pltpu.VMEM((CH, 8), F32),
    ],
)(_attn_body)


BMM = 400
_ACC_PAD = 15 * RPT + 16 * BMM


def _msg_body(src_hbm, dst_hbm, w_hbm, hw4_hbm, outp_hbm,
              acc_s, src_c, dst_b, w_c, rows_v):
    cid = lax.axis_index("c")
    sid = lax.axis_index("s")
    wid = sid * NC + cid
    ridx = lax.iota(jnp.int32, 16) // 4
    cidx = lax.iota(jnp.int32, 16) % 4
    zero16 = jnp.zeros((16,), F32)

    for h in range(4):
        def _zz(j, c):
            rows_v[j, :] = zero16
            return c
        lax.fori_loop(0, BMM, _zz, 0)
        for t in range(16):
            pltpu.sync_copy(rows_v, acc_s.at[pl.ds(sid * RPT + t * BMM, BMM), :])
        plsc.subcore_barrier()

        def _chunk(q, c, _h=h):
            eb = wid * EPT + q * CH
            pltpu.sync_copy(src_hbm.at[pl.ds(eb, CH)], src_c)
            pltpu.sync_copy(w_hbm.at[pl.ds(eb, CH), :], w_c)

            def _mkidx(g, cc):
                sl = pl.ds(16 * g, 16)
                src_c[sl] = 4 * src_c[sl] + _h
                return cc
            lax.fori_loop(0, CH // 16, _mkidx, 0)

            for b in range(CH // BMM):
                pltpu.sync_copy(dst_hbm.at[pl.ds(eb + b * BMM, BMM)], dst_b)
                pltpu.sync_copy(hw4_hbm.at[src_c.at[pl.ds(b * BMM, BMM)]],
                                rows_v)

                def _cmp(g, cc, _b=b, _hh=_h):
                    for u in range(4):
                        gg = 4 * g + u
                        w16 = plsc.load_gather(
                            w_c, [_b * BMM + ridx + 4 * gg, cidx])
                        for k in range(4):
                            j = 4 * gg + k
                            spl = _take16(w16,
                                          jnp.full((16,), 4 * k + _hh,
                                                   jnp.int32))
                            rows_v[j, :] = rows_v[j, :] * spl
                    return cc
                lax.fori_loop(0, BMM // 16, _cmp, 0)

                pltpu.sync_copy(rows_v, acc_s.at[dst_b], add=True)
            return c
        lax.fori_loop(0, EPT // CH, _chunk, 0)

        plsc.subcore_barrier()
        fst = pl.multiple_of(sid * RPT - 2 * (sid % 4), 8)
        pltpu.sync_copy(acc_s.at[pl.ds(fst, RPT + 6), :],
                        outp_hbm.at[cid, h, pl.ds(fst, RPT + 6), :])
        if h < 3:
            plsc.subcore_barrier()


_msg = functools.partial(
    pl.kernel,
    out_type=jax.ShapeDtypeStruct((NC, 4, NN, 16), F32),
    mesh=_MESH,
    compiler_params=pltpu.CompilerParams(needs_layout_passes=False, use_tc_tiling_on_sc=False),
    scratch_types=[
        pltpu.VMEM_SHARED((_ACC_PAD, 16), F32),
        pltpu.VMEM((CH,), jnp.int32),
        pltpu.VMEM((BMM,), jnp.int32),
        pltpu.VMEM((CH, 8), F32),
        pltpu.VMEM((BMM, 16), F32),
    ],
)(_msg_body)


def _blockdiag2(w):
    k = w.shape[0]
    z = jnp.zeros((2 * k, 128), F32)
    return z.at[:k, :64].set(w).at[k:, 64:].set(w)


def _build_aed(a_s, a_d):
    base = jnp.stack([a_s, a_d], 0)
    t = jnp.einsum("thd,hk->hdtk", base, jnp.eye(4, dtype=F32))
    t64 = t.reshape(64, 2, 4)
    blk = jnp.zeros((64, 32), F32)
    blk = blk.at[:, 0:4].set(t64[:, 0]).at[:, 16:20].set(t64[:, 1])
    a = jnp.zeros((128, 64), F32)
    return a.at[:64, :32].set(blk).at[64:, 32:].set(blk)


def kernel(high_x, high_edge_index, low_x, low_edge_index, mlp_high_W,
           mlp_high_b, mlp_low_W, mlp_low_b, Wh, ash, adh, Wl, asl, adl,
           ln_gamma, ln_beta):
    e8 = jnp.concatenate([jnp.eye(4, dtype=F32), jnp.zeros((4, 4), F32)], 0)
    ex = jnp.kron(jnp.eye(2, dtype=F32), jnp.kron(e8, jnp.ones((1, 16), F32)))
    pst = jnp.stack([
        jnp.kron(jnp.eye(2, dtype=F32),
                 jnp.kron(jax.nn.one_hot(h, 4, dtype=F32)[None, :],
                          jnp.eye(16, dtype=F32)))
        for h in range(4)])
    m64 = jnp.kron(jnp.eye(2, dtype=F32), jnp.ones((64, 64), F32) / 64.0)
    g128 = jnp.tile(ln_gamma, 2)[None, :]
    b128 = jnp.tile(ln_beta, 2)[None, :]

    outs = []
    for x, eidx, w0, b0, wl, a_s, a_d in (
            (high_x, high_edge_index, mlp_high_W, mlp_high_b, Wh, ash, adh),
            (low_x, low_edge_index, mlp_low_W, mlp_low_b, Wl, asl, adl)):
        src, dst = eidx[0], eidx[1]
        x2 = x.reshape(NH, 2 * x.shape[1])
        bds = [_blockdiag2(wl[i]) for i in range(4)]
        aeds = [_build_aed(a_s[i], a_d[i]) for i in range(4)]
        emb, hw, esed = _tc_embed(x2, _blockdiag2(w0),
                                  jnp.tile(b0, 2)[None, :], bds[0], aeds[0])
        for i in range(4):
            w, denp = _attn_w(src, dst, esed.reshape(2 * NN, 16))
            outp = _msg(src, dst, w, hw.reshape(4 * NN, 16))
            op = outp.reshape(NC, 4, NH, 32)
            dp = denp.reshape(NC, NH, 16)
            if i < 3:
                emb, hw, esed = _tc_mid(op, dp, emb, pst, ex,
                                        bds[i + 1], aeds[i + 1],
                                        residual=(i > 0))
            else:
                y = _tc_final(op, dp, emb, pst, ex, m64, g128, b128)
        outs.append(y.reshape(NN, 64))
    return tuple(outs)
